# baseline (speedup 1.0000x reference)
import jax
import jax.numpy as jnp
from jax.experimental import pallas as pl
from jax.experimental.pallas import tpu as pltpu

_CHUNKS = 8


def _copy_kernel(in_ref, out_ref, sems):
    rows = in_ref.shape[0]
    per = rows // _CHUNKS
    copies = []
    for i in range(_CHUNKS):
        sl = pl.ds(i * per, per)
        c = pltpu.make_async_copy(in_ref.at[sl], out_ref.at[sl], sems.at[i])
        c.start()
        copies.append(c)
    for c in copies:
        c.wait()


def kernel(samples, sample_rate):
    x = samples.reshape(samples.shape[0], -1)
    out = pl.pallas_call(
        _copy_kernel,
        out_shape=jax.ShapeDtypeStruct(x.shape, x.dtype),
        in_specs=[pl.BlockSpec(memory_space=pltpu.MemorySpace.HBM)],
        out_specs=pl.BlockSpec(memory_space=pltpu.MemorySpace.HBM),
        scratch_shapes=[pltpu.SemaphoreType.DMA((_CHUNKS,))],
    )(x)
    return out.reshape(samples.shape)


# speedup vs baseline: 12.3532x; 12.3532x over previous
import jax
import jax.numpy as jnp
from jax.experimental import pallas as pl
from jax.experimental.pallas import tpu as pltpu

_BLOCK_ROWS = 8


def _copy_kernel(in_ref, out_ref):
    out_ref[...] = in_ref[...]


def kernel(samples, sample_rate):
    x = samples.reshape(samples.shape[0], -1)
    rows, cols = x.shape
    grid = (rows // _BLOCK_ROWS,)
    out = pl.pallas_call(
        _copy_kernel,
        out_shape=jax.ShapeDtypeStruct(x.shape, x.dtype),
        grid=grid,
        in_specs=[pl.BlockSpec((_BLOCK_ROWS, cols), lambda i: (i, 0))],
        out_specs=pl.BlockSpec((_BLOCK_ROWS, cols), lambda i: (i, 0)),
    )(x)
    return out.reshape(samples.shape)


# speedup vs baseline: 12.4265x; 1.0059x over previous
import jax
import jax.numpy as jnp
from jax.experimental import pallas as pl
from jax.experimental.pallas import tpu as pltpu

_CHUNK_ROWS = 2
_N_CHUNKS = 32
_SLOTS = 16
_W = 8


def _copy_kernel(in_hbm, out_hbm, buf, in_sems, out_sems):
    def in_copy(c):
        return pltpu.make_async_copy(
            in_hbm.at[pl.ds(c * _CHUNK_ROWS, _CHUNK_ROWS)],
            buf.at[c % _SLOTS],
            in_sems.at[c % _SLOTS])

    def out_copy(c):
        return pltpu.make_async_copy(
            buf.at[c % _SLOTS],
            out_hbm.at[pl.ds(c * _CHUNK_ROWS, _CHUNK_ROWS)],
            out_sems.at[c % _SLOTS])

    for c in range(_W):
        in_copy(c).start()
    for c in range(_N_CHUNKS):
        in_copy(c).wait()
        out_copy(c).start()
        n = c + _W
        if n < _N_CHUNKS:
            if n >= _SLOTS:
                out_copy(n - _SLOTS).wait()
            in_copy(n).start()
    for c in range(max(0, _N_CHUNKS - _SLOTS), _N_CHUNKS):
        out_copy(c).wait()


def kernel(samples, sample_rate):
    x = samples.reshape(samples.shape[0], -1)
    rows, cols = x.shape
    out = pl.pallas_call(
        _copy_kernel,
        out_shape=jax.ShapeDtypeStruct(x.shape, x.dtype),
        in_specs=[pl.BlockSpec(memory_space=pltpu.MemorySpace.HBM)],
        out_specs=pl.BlockSpec(memory_space=pltpu.MemorySpace.HBM),
        scratch_shapes=[
            pltpu.VMEM((_SLOTS, _CHUNK_ROWS, cols), x.dtype),
            pltpu.SemaphoreType.DMA((_SLOTS,)),
            pltpu.SemaphoreType.DMA((_SLOTS,)),
        ],
    )(x)
    return out.reshape(samples.shape)


# speedup vs baseline: 51.1057x; 4.1126x over previous
import jax
import jax.numpy as jnp
from jax.experimental import pallas as pl
from jax.experimental.pallas import tpu as pltpu

_LANES = 128
_N_CHUNKS = 32
_SLOTS = 16
_W = 8


def _copy_kernel(chunk_rows, in_hbm, out_hbm, buf, in_sems, out_sems):
    def in_copy(c):
        return pltpu.make_async_copy(
            in_hbm.at[pl.ds(c * chunk_rows, chunk_rows)],
            buf.at[c % _SLOTS],
            in_sems.at[c % _SLOTS])

    def out_copy(c):
        return pltpu.make_async_copy(
            buf.at[c % _SLOTS],
            out_hbm.at[pl.ds(c * chunk_rows, chunk_rows)],
            out_sems.at[c % _SLOTS])

    for c in range(_W):
        in_copy(c).start()
    for c in range(_N_CHUNKS):
        in_copy(c).wait()
        out_copy(c).start()
        n = c + _W
        if n < _N_CHUNKS:
            if n >= _SLOTS:
                out_copy(n - _SLOTS).wait()
            in_copy(n).start()
    for c in range(max(0, _N_CHUNKS - _SLOTS), _N_CHUNKS):
        out_copy(c).wait()


def kernel(samples, sample_rate):
    rows = samples.size // _LANES
    chunk_rows = rows // _N_CHUNKS
    x = samples.reshape(rows, _LANES)
    out = pl.pallas_call(
        lambda *a: _copy_kernel(chunk_rows, *a),
        out_shape=jax.ShapeDtypeStruct(x.shape, x.dtype),
        in_specs=[pl.BlockSpec(memory_space=pltpu.MemorySpace.HBM)],
        out_specs=pl.BlockSpec(memory_space=pltpu.MemorySpace.HBM),
        scratch_shapes=[
            pltpu.VMEM((_SLOTS, chunk_rows, _LANES), x.dtype),
            pltpu.SemaphoreType.DMA((_SLOTS,)),
            pltpu.SemaphoreType.DMA((_SLOTS,)),
        ],
    )(x)
    return out.reshape(samples.shape)
